# baseline (speedup 1.0000x reference)
import functools
import math

import jax
import jax.numpy as jnp
from jax import lax
from jax.experimental import pallas as pl
from jax.experimental.pallas import tpu as pltpu
from jax.experimental.pallas import tpu_sc as plsc

INPUT_DIM = 64
NUM_PROJS = 7
NUM_BUCKETS = 1 << NUM_PROJS
BLOCK_SIZE = 256
SAMPLE_SIZE = 256
N_SEQ = 8192
NUM_BLOCKS = N_SEQ // BLOCK_SIZE
RANK_CHUNK = 256


def _hash_rank_body(q_ref, k_ref, pd_ref, posq_ref, posk_ref):
    pd = pd_ref[...]
    lane = lax.broadcasted_iota(jnp.int32, (N_SEQ, NUM_BUCKETS), 1)
    enc = jnp.where(lane < NUM_PROJS, 1 << jnp.minimum(lane, NUM_PROJS - 1), 0)
    r = lax.broadcasted_iota(jnp.int32, (RANK_CHUNK, RANK_CHUNK), 0)
    c = lax.broadcasted_iota(jnp.int32, (RANK_CHUNK, RANK_CHUNK), 1)
    L_incl = (c <= r).astype(jnp.float32)
    br = lax.broadcasted_iota(jnp.int32, (NUM_BUCKETS, NUM_BUCKETS), 0)
    bc = lax.broadcasted_iota(jnp.int32, (NUM_BUCKETS, NUM_BUCKETS), 1)
    SU = (br < bc).astype(jnp.float32)

    def rank_of(x):
        proj = jax.lax.dot_general(x, pd, (((1,), (0,)), ((), ())),
                                   preferred_element_type=jnp.float32)
        bits = jnp.where((proj > 0) & (lane < NUM_PROJS), enc, 0)
        binv = jnp.sum(bits, axis=1, keepdims=True)
        h = binv ^ (binv >> 1)
        oh = (h == lane).astype(jnp.float32)
        hist = jnp.sum(oh, axis=0, keepdims=True)
        bs = jax.lax.dot_general(hist, SU, (((1,), (0,)), ((), ())),
                                 preferred_element_type=jnp.float32)

        def chunk(i, carry):
            ohc = oh[i * RANK_CHUNK:(i + 1) * RANK_CHUNK, :]
            incl = jax.lax.dot_general(L_incl, ohc, (((1,), (0,)), ((), ())),
                                       preferred_element_type=jnp.float32)
            posc = jnp.sum(ohc * (bs + carry + incl), axis=1) - 1.0
            carry = carry + jnp.sum(ohc, axis=0, keepdims=True)
            return posc.astype(jnp.int32), carry

        carry = jnp.zeros((1, NUM_BUCKETS), jnp.float32)
        pieces = []
        for i in range(N_SEQ // RANK_CHUNK):
            posc, carry = chunk(i, carry)
            pieces.append(posc)
        return jnp.concatenate(pieces, axis=0)

    posq_ref[0, 0] = rank_of(q_ref[0])
    posk_ref[0, 0] = rank_of(k_ref[0])


def _hash_rank(q2, k2, proj_pad):
    BH = q2.shape[0]
    qspec = pl.BlockSpec((1, N_SEQ, INPUT_DIM), lambda i: (i, 0, 0))
    pspec = pl.BlockSpec((INPUT_DIM, NUM_BUCKETS), lambda i: (0, 0))
    ospec = pl.BlockSpec((1, 1, N_SEQ), lambda i: (i, 0, 0))
    pos_q, pos_k = pl.pallas_call(
        _hash_rank_body,
        grid=(BH,),
        in_specs=[qspec, qspec, pspec],
        out_specs=[ospec, ospec],
        out_shape=[jax.ShapeDtypeStruct((BH, 1, N_SEQ), jnp.int32),
                   jax.ShapeDtypeStruct((BH, 1, N_SEQ), jnp.int32)],
    )(q2, k2, proj_pad)
    return pos_q.reshape(BH, N_SEQ), pos_k.reshape(BH, N_SEQ)


def _attn_body(q_ref, kv_ref, sub_ref, samp_ref, out_ref):
    nb = pl.program_id(1)
    scale = INPUT_DIM ** (-0.5)
    qb = q_ref[0, 0]
    kvb = kv_ref[0, 0]
    kb = kvb[:, :INPUT_DIM]
    vb = kvb[:, INPUT_DIM:]
    sub = sub_ref[0]
    ks = sub[:, :INPUT_DIM]
    vs = sub[:, INPUT_DIM:]
    samp = samp_ref[0, 0]

    s1 = jax.lax.dot_general(qb, kb, (((1,), (1,)), ((), ())),
                             preferred_element_type=jnp.float32) * scale
    m1 = jnp.max(s1, axis=1, keepdims=True)
    p1 = jnp.exp(s1 - m1)
    l1 = jnp.sum(p1, axis=1, keepdims=True)
    a1 = jax.lax.dot_general(p1, vb, (((1,), (0,)), ((), ())),
                             preferred_element_type=jnp.float32)
    lse1 = m1 + jnp.log(l1)

    s2 = jax.lax.dot_general(qb, ks, (((1,), (1,)), ((), ())),
                             preferred_element_type=jnp.float32) * scale
    blk_of_samp = samp // BLOCK_SIZE
    neg = jnp.float32(jnp.finfo(jnp.float32).min)
    bias = jnp.where(blk_of_samp == nb, neg, jnp.float32(0.0))[None, :]
    s2 = s2 + bias
    m2 = jnp.max(s2, axis=1, keepdims=True)
    p2 = jnp.exp(s2 - m2)
    l2 = jnp.sum(p2, axis=1, keepdims=True)
    a2 = jax.lax.dot_general(p2, vs, (((1,), (0,)), ((), ())),
                             preferred_element_type=jnp.float32)
    lse2 = m2 + jnp.log(l2) + jnp.float32(math.log(N_SEQ / SAMPLE_SIZE))

    c = jax.nn.sigmoid(lse1 - lse2)
    out = c * (a1 / l1) + (1.0 - c) * (a2 / l2)
    out_ref[0, 0] = out


def _fused_attention(q_sorted, kv_sorted, kv_sub, samp):
    BH, N, D = q_sorted.shape
    nb = NUM_BLOCKS
    qs4 = q_sorted.reshape(BH, nb, BLOCK_SIZE, D)
    kvs4 = kv_sorted.reshape(BH, nb, BLOCK_SIZE, 2 * D)
    grid = (BH, nb)
    blk = pl.BlockSpec((1, 1, BLOCK_SIZE, D), lambda i, j: (i, j, 0, 0))
    kvblk = pl.BlockSpec((1, 1, BLOCK_SIZE, 2 * D), lambda i, j: (i, j, 0, 0))
    sub = pl.BlockSpec((1, SAMPLE_SIZE, 2 * D), lambda i, j: (i, 0, 0))
    sspec = pl.BlockSpec((1, 1, SAMPLE_SIZE), lambda i, j: (i, 0, 0))
    out = pl.pallas_call(
        _attn_body,
        grid=grid,
        in_specs=[blk, kvblk, sub, sspec],
        out_specs=blk,
        out_shape=jax.ShapeDtypeStruct((BH, nb, BLOCK_SIZE, D), jnp.float32),
    )(qs4, kvs4, kv_sub, samp)
    return out.reshape(BH, N, D)


def kernel(query, key, value, proj_dir, sampled_set):
    B, H, N, D = query.shape
    BH = B * H
    q2 = query.reshape(BH, N, D)
    k2 = key.reshape(BH, N, D)
    v2 = value.reshape(BH, N, D)
    samp2 = sampled_set.reshape(BH, SAMPLE_SIZE)

    proj_pad = jnp.zeros((INPUT_DIM, NUM_BUCKETS), jnp.float32)
    proj_pad = proj_pad.at[:, :NUM_PROJS].set(proj_dir[:INPUT_DIM])

    pos_q, pos_k = _hash_rank(q2, k2, proj_pad)

    kv2 = jnp.concatenate([k2, v2], axis=-1)
    scat = jax.vmap(lambda x, p: jnp.zeros_like(x).at[p].set(
        x, unique_indices=True, mode="promise_in_bounds"))
    q_sorted = scat(q2, pos_q)
    kv_sorted = scat(kv2, pos_k)

    kv_sub = jnp.take_along_axis(kv_sorted, samp2[..., None], axis=1)

    merged = _fused_attention(q_sorted, kv_sorted, kv_sub,
                              samp2.reshape(BH, 1, SAMPLE_SIZE))

    out = jnp.take_along_axis(merged, pos_q[..., None], axis=1)
    return out.reshape(B, H, N, D)


# speedup vs baseline: 3.3938x; 3.3938x over previous
import functools
import math

import jax
import jax.numpy as jnp
from jax import lax
from jax.experimental import pallas as pl
from jax.experimental.pallas import tpu as pltpu
from jax.experimental.pallas import tpu_sc as plsc

INPUT_DIM = 64
NUM_PROJS = 7
NUM_BUCKETS = 1 << NUM_PROJS
BLOCK_SIZE = 256
SAMPLE_SIZE = 256
N_SEQ = 8192
NUM_BLOCKS = N_SEQ // BLOCK_SIZE
RANK_CHUNK = 256


def _hash_rank_body(q_ref, k_ref, v_ref, pd_ref, posq_ref, posk_ref,
                    qpad_ref, kv_ref):
    pd = pd_ref[...]
    lane = lax.broadcasted_iota(jnp.int32, (N_SEQ, NUM_BUCKETS), 1)
    enc = jnp.where(lane < NUM_PROJS, 1 << jnp.minimum(lane, NUM_PROJS - 1), 0)
    r = lax.broadcasted_iota(jnp.int32, (RANK_CHUNK, RANK_CHUNK), 0)
    c = lax.broadcasted_iota(jnp.int32, (RANK_CHUNK, RANK_CHUNK), 1)
    L_incl = (c <= r).astype(jnp.float32)
    br = lax.broadcasted_iota(jnp.int32, (NUM_BUCKETS, NUM_BUCKETS), 0)
    bc = lax.broadcasted_iota(jnp.int32, (NUM_BUCKETS, NUM_BUCKETS), 1)
    SU = (br < bc).astype(jnp.float32)

    def rank_of(x):
        proj = jax.lax.dot_general(x, pd, (((1,), (0,)), ((), ())),
                                   preferred_element_type=jnp.float32)
        bits = jnp.where((proj > 0) & (lane < NUM_PROJS), enc, 0)
        binv = jnp.sum(bits, axis=1, keepdims=True)
        h = binv ^ (binv >> 1)
        oh = (h == lane).astype(jnp.float32)
        hist = jnp.sum(oh, axis=0, keepdims=True)
        bs = jax.lax.dot_general(hist, SU, (((1,), (0,)), ((), ())),
                                 preferred_element_type=jnp.float32)

        def chunk(i, carry):
            ohc = oh[i * RANK_CHUNK:(i + 1) * RANK_CHUNK, :]
            incl = jax.lax.dot_general(L_incl, ohc, (((1,), (0,)), ((), ())),
                                       preferred_element_type=jnp.float32)
            posc = jnp.sum(ohc * (bs + carry + incl), axis=1) - 1.0
            carry = carry + jnp.sum(ohc, axis=0, keepdims=True)
            return posc.astype(jnp.int32), carry

        carry = jnp.zeros((1, NUM_BUCKETS), jnp.float32)
        pieces = []
        for i in range(N_SEQ // RANK_CHUNK):
            posc, carry = chunk(i, carry)
            pieces.append(posc)
        return jnp.concatenate(pieces, axis=0)

    base = pl.program_id(0) * N_SEQ
    posq_ref[0, 0] = rank_of(q_ref[0]) + base
    posk_ref[0, 0] = rank_of(k_ref[0]) + base
    zpad = jnp.zeros((N_SEQ, INPUT_DIM), jnp.float32)
    qpad_ref[0] = jnp.concatenate([q_ref[0], zpad], axis=1)
    kv_ref[0] = jnp.concatenate([k_ref[0], v_ref[0]], axis=1)


def _hash_rank(q2, k2, v2, proj_pad):
    BH = q2.shape[0]
    qspec = pl.BlockSpec((1, N_SEQ, INPUT_DIM), lambda i: (i, 0, 0))
    pspec = pl.BlockSpec((INPUT_DIM, NUM_BUCKETS), lambda i: (0, 0))
    ospec = pl.BlockSpec((1, 1, N_SEQ), lambda i: (i, 0, 0))
    wspec = pl.BlockSpec((1, N_SEQ, 2 * INPUT_DIM), lambda i: (i, 0, 0))
    pos_q, pos_k, qpad, kv = pl.pallas_call(
        _hash_rank_body,
        grid=(BH,),
        in_specs=[qspec, qspec, qspec, pspec],
        out_specs=[ospec, ospec, wspec, wspec],
        out_shape=[jax.ShapeDtypeStruct((BH, 1, N_SEQ), jnp.int32),
                   jax.ShapeDtypeStruct((BH, 1, N_SEQ), jnp.int32),
                   jax.ShapeDtypeStruct((BH, N_SEQ, 2 * INPUT_DIM),
                                        jnp.float32),
                   jax.ShapeDtypeStruct((BH, N_SEQ, 2 * INPUT_DIM),
                                        jnp.float32)],
    )(q2, k2, v2, proj_pad)
    return pos_q.reshape(BH, N_SEQ), pos_k.reshape(BH, N_SEQ), qpad, kv


def _attn_body(q_ref, kv_ref, sub_ref, samp_ref, out_ref):
    nb = pl.program_id(1)
    scale = INPUT_DIM ** (-0.5)
    qb = q_ref[0, 0][:, :INPUT_DIM]
    kvb = kv_ref[0, 0]
    kb = kvb[:, :INPUT_DIM]
    vb = kvb[:, INPUT_DIM:]
    sub = sub_ref[0]
    ks = sub[:, :INPUT_DIM]
    vs = sub[:, INPUT_DIM:]
    samp = samp_ref[0, 0]

    s1 = jax.lax.dot_general(qb, kb, (((1,), (1,)), ((), ())),
                             preferred_element_type=jnp.float32) * scale
    m1 = jnp.max(s1, axis=1, keepdims=True)
    p1 = jnp.exp(s1 - m1)
    l1 = jnp.sum(p1, axis=1, keepdims=True)
    a1 = jax.lax.dot_general(p1, vb, (((1,), (0,)), ((), ())),
                             preferred_element_type=jnp.float32)
    lse1 = m1 + jnp.log(l1)

    s2 = jax.lax.dot_general(qb, ks, (((1,), (1,)), ((), ())),
                             preferred_element_type=jnp.float32) * scale
    blk_of_samp = samp // BLOCK_SIZE
    neg = jnp.float32(jnp.finfo(jnp.float32).min)
    bias = jnp.where(blk_of_samp == nb, neg, jnp.float32(0.0))[None, :]
    s2 = s2 + bias
    m2 = jnp.max(s2, axis=1, keepdims=True)
    p2 = jnp.exp(s2 - m2)
    l2 = jnp.sum(p2, axis=1, keepdims=True)
    a2 = jax.lax.dot_general(p2, vs, (((1,), (0,)), ((), ())),
                             preferred_element_type=jnp.float32)
    lse2 = m2 + jnp.log(l2) + jnp.float32(math.log(N_SEQ / SAMPLE_SIZE))

    c = jax.nn.sigmoid(lse1 - lse2)
    out = c * (a1 / l1) + (1.0 - c) * (a2 / l2)
    out_ref[0, 0] = out


SUPER = 256
NSUP = N_SEQ // SUPER
IDXW = 128


def _make_permute_kernel(BH):
    info = plsc.get_sparse_core_info()
    NC = info.num_cores
    mesh = plsc.VectorSubcoreMesh(core_axis_name="c", subcore_axis_name="s")
    BHN = BH * N_SEQ
    S = SAMPLE_SIZE
    W = 2 * INPUT_DIM

    @functools.partial(
        pl.kernel,
        out_type=[jax.ShapeDtypeStruct((BHN, W), jnp.float32),
                  jax.ShapeDtypeStruct((BHN, W), jnp.float32),
                  jax.ShapeDtypeStruct((BH * S, W), jnp.float32)],
        mesh=mesh,
        scratch_types=[pltpu.VMEM((N_SEQ // IDXW, IDXW), jnp.int32),
                       pltpu.VMEM((SUPER, W), jnp.float32),
                       pltpu.VMEM((S // IDXW, IDXW), jnp.int32),
                       pltpu.VMEM((S, W), jnp.float32),
                       pltpu.SemaphoreType.DMA],
        compiler_params=pltpu.CompilerParams(needs_layout_passes=False),
    )
    def permute(qpad_hbm, kv_hbm, posq_hbm, posk_hbm, samp_hbm,
                qs_hbm, kvs_hbm, sub_hbm,
                idx_v, buf_v, sidx_v, sub_v, sem):
        wid = lax.axis_index("s") * NC + lax.axis_index("c")
        base = wid * N_SEQ
        per = SUPER // IDXW

        def scatter_tensor(src_hbm, dst_hbm):
            def step(s, _):
                pltpu.sync_copy(src_hbm.at[pl.ds(base + s * SUPER, SUPER)],
                                buf_v)
                for p in range(per):
                    pltpu.async_copy(
                        buf_v.at[pl.ds(p * IDXW, IDXW)],
                        dst_hbm.at[idx_v.at[s * per + p]], sem).wait()
                return 0
            lax.fori_loop(0, NSUP, step, 0)

        pltpu.sync_copy(posq_hbm.at[wid], idx_v)
        scatter_tensor(qpad_hbm, qs_hbm)
        pltpu.sync_copy(posk_hbm.at[wid], idx_v)
        scatter_tensor(kv_hbm, kvs_hbm)

        pltpu.sync_copy(samp_hbm.at[wid], sidx_v)
        for p in range(S // IDXW):
            pltpu.async_copy(kvs_hbm.at[sidx_v.at[p]],
                             sub_v.at[pl.ds(p * IDXW, IDXW)], sem).wait()
        pltpu.sync_copy(sub_v, sub_hbm.at[pl.ds(wid * S, S)])

    return permute


def _fused_attention(qs_pad, kv_sorted, kv_sub, samp):
    BH = qs_pad.shape[0]
    D = INPUT_DIM
    nb = NUM_BLOCKS
    qs4 = qs_pad.reshape(BH, nb, BLOCK_SIZE, 2 * D)
    kvs4 = kv_sorted.reshape(BH, nb, BLOCK_SIZE, 2 * D)
    grid = (BH, nb)
    oblk = pl.BlockSpec((1, 1, BLOCK_SIZE, D), lambda i, j: (i, j, 0, 0))
    kvblk = pl.BlockSpec((1, 1, BLOCK_SIZE, 2 * D), lambda i, j: (i, j, 0, 0))
    sub = pl.BlockSpec((1, SAMPLE_SIZE, 2 * D), lambda i, j: (i, 0, 0))
    sspec = pl.BlockSpec((1, 1, SAMPLE_SIZE), lambda i, j: (i, 0, 0))
    out = pl.pallas_call(
        _attn_body,
        grid=grid,
        in_specs=[kvblk, kvblk, sub, sspec],
        out_specs=oblk,
        out_shape=jax.ShapeDtypeStruct((BH, nb, BLOCK_SIZE, D), jnp.float32),
    )(qs4, kvs4, kv_sub, samp)
    return out.reshape(BH, N_SEQ, D)


def kernel(query, key, value, proj_dir, sampled_set):
    B, H, N, D = query.shape
    BH = B * H
    q2 = query.reshape(BH, N, D)
    k2 = key.reshape(BH, N, D)
    v2 = value.reshape(BH, N, D)
    samp2 = sampled_set.reshape(BH, SAMPLE_SIZE)

    proj_pad = jnp.zeros((INPUT_DIM, NUM_BUCKETS), jnp.float32)
    proj_pad = proj_pad.at[:, :NUM_PROJS].set(proj_dir[:INPUT_DIM])

    pos_q, pos_k, qpad, kv = _hash_rank(q2, k2, v2, proj_pad)

    offs = jnp.arange(BH, dtype=jnp.int32)[:, None] * N
    samp_g = (samp2 + offs).reshape(BH, SAMPLE_SIZE // IDXW, IDXW)
    posq3 = pos_q.reshape(BH, N // IDXW, IDXW)
    posk3 = pos_k.reshape(BH, N // IDXW, IDXW)

    qs, kvs, sub = _make_permute_kernel(BH)(
        qpad.reshape(BH * N, 2 * D), kv.reshape(BH * N, 2 * D),
        posq3, posk3, samp_g)

    merged = _fused_attention(qs.reshape(BH, N, 2 * D),
                              kvs.reshape(BH, N, 2 * D),
                              sub.reshape(BH, SAMPLE_SIZE, 2 * D),
                              samp2.reshape(BH, 1, SAMPLE_SIZE))

    out = jnp.take(merged.reshape(BH * N, D), pos_q.reshape(BH * N), axis=0)
    return out.reshape(B, H, N, D)


# speedup vs baseline: 3.9802x; 1.1728x over previous
import functools
import math

import jax
import jax.numpy as jnp
from jax import lax
from jax.experimental import pallas as pl
from jax.experimental.pallas import tpu as pltpu
from jax.experimental.pallas import tpu_sc as plsc

INPUT_DIM = 64
NUM_PROJS = 7
NUM_BUCKETS = 1 << NUM_PROJS
BLOCK_SIZE = 256
SAMPLE_SIZE = 256
N_SEQ = 8192
NUM_BLOCKS = N_SEQ // BLOCK_SIZE
RANK_CHUNK = 256


def _hash_rank_body(q_ref, k_ref, v_ref, pd_ref, posq_ref, posk_ref,
                    qpad_ref, kv_ref):
    pd = pd_ref[...]
    lane = lax.broadcasted_iota(jnp.int32, (N_SEQ, NUM_BUCKETS), 1)
    enc = jnp.where(lane < NUM_PROJS, 1 << jnp.minimum(lane, NUM_PROJS - 1), 0)
    r = lax.broadcasted_iota(jnp.int32, (RANK_CHUNK, RANK_CHUNK), 0)
    c = lax.broadcasted_iota(jnp.int32, (RANK_CHUNK, RANK_CHUNK), 1)
    L_incl = (c <= r).astype(jnp.bfloat16)
    br = lax.broadcasted_iota(jnp.int32, (NUM_BUCKETS, NUM_BUCKETS), 0)
    bc = lax.broadcasted_iota(jnp.int32, (NUM_BUCKETS, NUM_BUCKETS), 1)
    SU = (br < bc).astype(jnp.float32)

    def rank_of(x):
        proj = jax.lax.dot_general(x, pd, (((1,), (0,)), ((), ())),
                                   preferred_element_type=jnp.float32)
        bits = jnp.where((proj > 0) & (lane < NUM_PROJS), enc, 0)
        binv = jnp.sum(bits, axis=1, keepdims=True)
        h = binv ^ (binv >> 1)
        ohb = (h == lane).astype(jnp.bfloat16)
        oh = ohb.astype(jnp.float32)
        hist = jnp.sum(oh, axis=0, keepdims=True)
        bs = jax.lax.dot_general(hist, SU, (((1,), (0,)), ((), ())),
                                 preferred_element_type=jnp.float32)

        def chunk(i, carry):
            ohc = oh[i * RANK_CHUNK:(i + 1) * RANK_CHUNK, :]
            ohcb = ohb[i * RANK_CHUNK:(i + 1) * RANK_CHUNK, :]
            incl = jax.lax.dot_general(L_incl, ohcb, (((1,), (0,)), ((), ())),
                                       preferred_element_type=jnp.float32)
            posc = jnp.sum(ohc * (bs + carry + incl), axis=1) - 1.0
            carry = carry + jnp.sum(ohc, axis=0, keepdims=True)
            return posc.astype(jnp.int32), carry

        carry = jnp.zeros((1, NUM_BUCKETS), jnp.float32)
        pieces = []
        for i in range(N_SEQ // RANK_CHUNK):
            posc, carry = chunk(i, carry)
            pieces.append(posc)
        return jnp.concatenate(pieces, axis=0)

    base = pl.program_id(0) * N_SEQ
    posq_ref[0, 0] = rank_of(q_ref[0]) + base
    posk_ref[0, 0] = rank_of(k_ref[0]) + base
    zpad = jnp.zeros((N_SEQ, INPUT_DIM), jnp.float32)
    qpad_ref[0] = jnp.concatenate([q_ref[0], zpad], axis=1)
    kv_ref[0] = jnp.concatenate([k_ref[0], v_ref[0]], axis=1)


def _hash_rank(q2, k2, v2, proj_pad):
    BH = q2.shape[0]
    qspec = pl.BlockSpec((1, N_SEQ, INPUT_DIM), lambda i: (i, 0, 0))
    pspec = pl.BlockSpec((INPUT_DIM, NUM_BUCKETS), lambda i: (0, 0))
    ospec = pl.BlockSpec((1, 1, N_SEQ), lambda i: (i, 0, 0))
    wspec = pl.BlockSpec((1, N_SEQ, 2 * INPUT_DIM), lambda i: (i, 0, 0))
    pos_q, pos_k, qpad, kv = pl.pallas_call(
        _hash_rank_body,
        grid=(BH,),
        in_specs=[qspec, qspec, qspec, pspec],
        out_specs=[ospec, ospec, wspec, wspec],
        out_shape=[jax.ShapeDtypeStruct((BH, 1, N_SEQ), jnp.int32),
                   jax.ShapeDtypeStruct((BH, 1, N_SEQ), jnp.int32),
                   jax.ShapeDtypeStruct((BH, N_SEQ, 2 * INPUT_DIM),
                                        jnp.float32),
                   jax.ShapeDtypeStruct((BH, N_SEQ, 2 * INPUT_DIM),
                                        jnp.float32)],
    )(q2, k2, v2, proj_pad)
    return pos_q.reshape(BH, N_SEQ), pos_k.reshape(BH, N_SEQ), qpad, kv


BLOCKS_PER_STEP = 2


def _attn_body(q_ref, kv_ref, sub_ref, samp_ref, out_ref):
    scale = INPUT_DIM ** (-0.5)
    sub = sub_ref[0]
    ks = sub[:, :INPUT_DIM]
    vs = sub[:, INPUT_DIM:]
    samp = samp_ref[0, 0]
    blk_of_samp = samp // BLOCK_SIZE
    neg = jnp.float32(jnp.finfo(jnp.float32).min)

    for t in range(BLOCKS_PER_STEP):
        nb = pl.program_id(1) * BLOCKS_PER_STEP + t
        qb = q_ref[0, t][:, :INPUT_DIM]
        kvb = kv_ref[0, t]
        kb = kvb[:, :INPUT_DIM]
        vb = kvb[:, INPUT_DIM:]

        s1 = jax.lax.dot_general(qb, kb, (((1,), (1,)), ((), ())),
                                 preferred_element_type=jnp.float32) * scale
        m1 = jnp.max(s1, axis=1, keepdims=True)
        p1 = jnp.exp(s1 - m1)
        l1 = jnp.sum(p1, axis=1, keepdims=True)
        a1 = jax.lax.dot_general(p1, vb, (((1,), (0,)), ((), ())),
                                 preferred_element_type=jnp.float32)
        lse1 = m1 + jnp.log(l1)

        s2 = jax.lax.dot_general(qb, ks, (((1,), (1,)), ((), ())),
                                 preferred_element_type=jnp.float32) * scale
        bias = jnp.where(blk_of_samp == nb, neg, jnp.float32(0.0))[None, :]
        s2 = s2 + bias
        m2 = jnp.max(s2, axis=1, keepdims=True)
        p2 = jnp.exp(s2 - m2)
        l2 = jnp.sum(p2, axis=1, keepdims=True)
        a2 = jax.lax.dot_general(p2, vs, (((1,), (0,)), ((), ())),
                                 preferred_element_type=jnp.float32)
        lse2 = m2 + jnp.log(l2) + jnp.float32(math.log(N_SEQ / SAMPLE_SIZE))

        c = jax.nn.sigmoid(lse1 - lse2)
        out = c * (a1 / l1) + (1.0 - c) * (a2 / l2)
        out_ref[0, t] = out


SUPER = 256
NSUP = N_SEQ // SUPER
IDXW = 128


def _make_permute_kernel(BH):
    info = plsc.get_sparse_core_info()
    NC = info.num_cores
    mesh = plsc.VectorSubcoreMesh(core_axis_name="c", subcore_axis_name="s")
    BHN = BH * N_SEQ
    S = SAMPLE_SIZE
    W = 2 * INPUT_DIM

    @functools.partial(
        pl.kernel,
        out_type=[jax.ShapeDtypeStruct((BHN, W), jnp.float32),
                  jax.ShapeDtypeStruct((BHN, W), jnp.float32),
                  jax.ShapeDtypeStruct((BH * S, W), jnp.float32)],
        mesh=mesh,
        scratch_types=[pltpu.VMEM((N_SEQ // IDXW, IDXW), jnp.int32),
                       pltpu.VMEM((SUPER, W), jnp.float32),
                       pltpu.VMEM((S // IDXW, IDXW), jnp.int32),
                       pltpu.VMEM((S, W), jnp.float32),
                       pltpu.SemaphoreType.DMA],
        compiler_params=pltpu.CompilerParams(needs_layout_passes=False),
    )
    def permute(qpad_hbm, kv_hbm, posq_hbm, posk_hbm, samp_hbm,
                qs_hbm, kvs_hbm, sub_hbm,
                idx_v, buf_v, sidx_v, sub_v, sem):
        wid = lax.axis_index("s") * NC + lax.axis_index("c")
        base = wid * N_SEQ
        per = SUPER // IDXW

        def scatter_tensor(src_hbm, dst_hbm):
            def step(s, _):
                pltpu.sync_copy(src_hbm.at[pl.ds(base + s * SUPER, SUPER)],
                                buf_v)
                for p in range(per):
                    pltpu.async_copy(
                        buf_v.at[pl.ds(p * IDXW, IDXW)],
                        dst_hbm.at[idx_v.at[s * per + p]], sem).wait()
                return 0
            lax.fori_loop(0, NSUP, step, 0)

        pltpu.sync_copy(posq_hbm.at[wid], idx_v)
        scatter_tensor(qpad_hbm, qs_hbm)
        pltpu.sync_copy(posk_hbm.at[wid], idx_v)
        scatter_tensor(kv_hbm, kvs_hbm)

        pltpu.sync_copy(samp_hbm.at[wid], sidx_v)
        for p in range(S // IDXW):
            pltpu.async_copy(kvs_hbm.at[sidx_v.at[p]],
                             sub_v.at[pl.ds(p * IDXW, IDXW)], sem).wait()
        pltpu.sync_copy(sub_v, sub_hbm.at[pl.ds(wid * S, S)])

    return permute


def _fused_attention(qs_pad, kv_sorted, kv_sub, samp):
    BH = qs_pad.shape[0]
    D = INPUT_DIM
    nb = NUM_BLOCKS
    qs4 = qs_pad.reshape(BH, nb, BLOCK_SIZE, 2 * D)
    kvs4 = kv_sorted.reshape(BH, nb, BLOCK_SIZE, 2 * D)
    grid = (BH, nb // BLOCKS_PER_STEP)
    oblk = pl.BlockSpec((1, BLOCKS_PER_STEP, BLOCK_SIZE, D),
                        lambda i, j: (i, j, 0, 0))
    kvblk = pl.BlockSpec((1, BLOCKS_PER_STEP, BLOCK_SIZE, 2 * D),
                         lambda i, j: (i, j, 0, 0))
    sub = pl.BlockSpec((1, SAMPLE_SIZE, 2 * D), lambda i, j: (i, 0, 0))
    sspec = pl.BlockSpec((1, 1, SAMPLE_SIZE), lambda i, j: (i, 0, 0))
    out = pl.pallas_call(
        _attn_body,
        grid=grid,
        in_specs=[kvblk, kvblk, sub, sspec],
        out_specs=oblk,
        out_shape=jax.ShapeDtypeStruct((BH, nb, BLOCK_SIZE, D), jnp.float32),
    )(qs4, kvs4, kv_sub, samp)
    return out.reshape(BH, N_SEQ, D)


def kernel(query, key, value, proj_dir, sampled_set):
    B, H, N, D = query.shape
    BH = B * H
    q2 = query.reshape(BH, N, D)
    k2 = key.reshape(BH, N, D)
    v2 = value.reshape(BH, N, D)
    samp2 = sampled_set.reshape(BH, SAMPLE_SIZE)

    proj_pad = jnp.zeros((INPUT_DIM, NUM_BUCKETS), jnp.float32)
    proj_pad = proj_pad.at[:, :NUM_PROJS].set(proj_dir[:INPUT_DIM])

    pos_q, pos_k, qpad, kv = _hash_rank(q2, k2, v2, proj_pad)

    offs = jnp.arange(BH, dtype=jnp.int32)[:, None] * N
    samp_g = (samp2 + offs).reshape(BH, SAMPLE_SIZE // IDXW, IDXW)
    posq3 = pos_q.reshape(BH, N // IDXW, IDXW)
    posk3 = pos_k.reshape(BH, N // IDXW, IDXW)

    qs, kvs, sub = _make_permute_kernel(BH)(
        qpad.reshape(BH * N, 2 * D), kv.reshape(BH * N, 2 * D),
        posq3, posk3, samp_g)

    merged = _fused_attention(qs.reshape(BH, N, 2 * D),
                              kvs.reshape(BH, N, 2 * D),
                              sub.reshape(BH, SAMPLE_SIZE, 2 * D),
                              samp2.reshape(BH, 1, SAMPLE_SIZE))

    out = jnp.take(merged.reshape(BH * N, D), pos_q.reshape(BH * N), axis=0)
    return out.reshape(B, H, N, D)
